# baseline (speedup 1.0000x reference)
import functools

import jax
import jax.numpy as jnp
import numpy as np
from jax import lax
from jax.experimental import pallas as pl
from jax.experimental.pallas import tpu as pltpu
from jax.experimental.pallas import tpu_sc as plsc

_MAP_SIZES = [64, 32, 16, 8, 4, 2]
_NB = 6
_B = 16
_NOBJ = 64
_NCLS = 21
_CP = 32
_LP = 8
_SEG_H = 256
_TOTAL = sum(_B * _NB * ms * ms for ms in _MAP_SIZES)
_CLS_RB = 5760
_CLS_STEPS = _TOTAL // _CLS_RB
_SEG_BH = 64
_SEG_STEPS = _B * (_SEG_H // _SEG_BH)

_LAYER_OFF = [0, 393216, 491520, 516096]
_LAYER_BSTRIDE = [24576, 6144, 1536, 384]

_CLS_ROWS = _TOTAL * _CP // 128
_LOC_ROWS = _TOTAL * _LP // 128
_CLS_RB2 = _CLS_ROWS // _CLS_STEPS

_M8 = np.zeros((128, 8), np.float32)
for _a in range(4):
    _M8[32 * _a:32 * _a + 32, _a] = 1.0
    _M8[32 * _a, 4 + _a] = 1.0


def _dense_body(x_ref, m_ref, seg_ref, lab_ref, acc_ref):
    i = pl.program_id(0)

    @pl.when(i == 0)
    def _():
        acc_ref[0, 0] = 0.0
        acc_ref[0, 1] = 0.0

    x = x_ref[...]
    e = jnp.exp(x).astype(jnp.bfloat16)
    y = jnp.dot(e, m_ref[...], preferred_element_type=jnp.float32)
    ly = jnp.log(y)
    acc_ref[0, 0] += jnp.sum(ly[:, :4]) - jnp.sum(ly[:, 4:])

    @pl.when(i < _SEG_STEPS)
    def _():
        lab = lab_ref[0]
        x0 = seg_ref[0, 0]
        se = jnp.exp(x0)
        xl = jnp.where(lab == 0, x0, 0.0)
        for c in range(1, _NCLS):
            xc = seg_ref[0, c]
            se = se + jnp.exp(xc)
            xl = jnp.where(lab == c, xc, xl)
        acc_ref[0, 1] += jnp.sum(jnp.log(se)) - jnp.sum(xl)


def _take16(x, idx):
    dnums = lax.GatherDimensionNumbers(
        offset_dims=(), collapsed_slice_dims=(0,), start_index_map=(0,))
    return lax.gather(x, idx[:, None], dnums, slice_sizes=(1,),
                      mode=lax.GatherScatterMode.PROMISE_IN_BOUNDS)


def _sc_body(clsp, locp, idxt, clsb, gtt, dft, out,
             liv, piv, biv, cbv, gtv, dfv, crows, lrows, outv, sem):
    w = lax.axis_index("s") * 2 + lax.axis_index("c")

    @pl.when(w < _B)
    def _():
        b = w
        pltpu.sync_copy(idxt.at[0, b], liv)
        pltpu.sync_copy(idxt.at[1, b], piv)
        pltpu.sync_copy(idxt.at[2, b], biv)
        pltpu.sync_copy(clsb.at[b], cbv)
        for c in range(4):
            pltpu.sync_copy(gtt.at[c, b], gtv.at[c])
            pltpu.sync_copy(dft.at[c, b], dfv.at[c])

        iota = lax.iota(jnp.int32, 16)
        flats = []
        labs = []
        handles = []
        for v in range(4):
            ly = liv[pl.ds(16 * v, 16)]
            ps = piv[pl.ds(16 * v, 16)]
            bx = biv[pl.ds(16 * v, 16)]
            lb = cbv[pl.ds(16 * v, 16)]
            off = jnp.where(
                ly == 0, _LAYER_OFF[0],
                jnp.where(ly == 1, _LAYER_OFF[1],
                          jnp.where(ly == 2, _LAYER_OFF[2], _LAYER_OFF[3])))
            bst = jnp.where(
                ly == 0, _LAYER_BSTRIDE[0],
                jnp.where(ly == 1, _LAYER_BSTRIDE[1],
                          jnp.where(ly == 2, _LAYER_BSTRIDE[2],
                                    _LAYER_BSTRIDE[3])))
            f = off + b * bst + ps * _NB + bx
            flats.append(f)
            labs.append(lb)
            handles.append(pltpu.async_copy(clsp.at[f >> 2], crows.at[v], sem))
            handles.append(pltpu.async_copy(locp.at[f >> 4], lrows.at[v], sem))

        wins = []
        for v in range(4):
            dup = jnp.zeros((16,), jnp.bool_)
            for k in range(1, 16):
                rolled = _take16(flats[v], (iota + k) & 15)
                dup = dup | ((rolled == flats[v]) & (iota < 16 - k))
            for u in range(v + 1, 4):
                for k in range(16):
                    rolled = _take16(flats[u], (iota + k) & 15)
                    dup = dup | (rolled == flats[v])
            wins.append(jnp.logical_not(dup))

        for h in handles:
            h.wait()

        cls_corr = jnp.float32(0.0)
        loc_sum = jnp.float32(0.0)
        npos = jnp.float32(0.0)
        for v in range(4):
            vvec = jnp.full((16,), v, jnp.int32)
            winf = wins[v].astype(jnp.float32)
            posf = (wins[v] & (labs[v] > 0)).astype(jnp.float32)
            claneb = (flats[v] & 3) * _CP
            llaneb = (flats[v] & 15) * _LP
            c0 = plsc.load_gather(crows, [vvec, iota, claneb])
            cl = plsc.load_gather(crows, [vvec, iota, claneb + labs[v]])
            cls_corr = cls_corr + jnp.sum((c0 - cl) * winf)
            sl1 = jnp.zeros((16,), jnp.float32)
            for c in range(4):
                gtc = gtv[c, pl.ds(16 * v, 16)]
                dfc = dfv[c, pl.ds(16 * v, 16)]
                lv = (gtc - dfc) / jnp.float32(0.1)
                lc = plsc.load_gather(lrows, [vvec, iota, llaneb + c])
                d = jnp.abs(lc - lv)
                sl1 = sl1 + jnp.where(d < 1.0, 0.5 * d * d, d - 0.5)
            loc_sum = loc_sum + jnp.sum(sl1 * posf)
            npos = npos + jnp.sum(posf)

        outv[...] = jnp.where(
            iota == 0, cls_corr,
            jnp.where(iota == 1, loc_sum,
                      jnp.where(iota == 2, npos, jnp.float32(0.0))))
        pltpu.sync_copy(outv, out.at[b])


def kernel(Loc, Cls, Seg, gt_box_batch, df_box_batch, idx_batch, cls_batch,
           bat_s, mining, seg_label):
    clsp = jnp.pad(Cls, ((0, 0), (0, _CP - _NCLS)),
                   constant_values=-1e30).reshape(_CLS_ROWS, 128)
    locp = jnp.pad(Loc, ((0, 0), (0, _LP - 4))).reshape(_LOC_ROWS, 128)

    def _seg_i(i):
        j = jnp.minimum(i, _SEG_STEPS - 1)
        return j // (_SEG_H // _SEG_BH), j % (_SEG_H // _SEG_BH)

    def _seg_map(i):
        bi, hi = _seg_i(i)
        return (bi, 0, hi, 0)

    def _lab_map(i):
        bi, hi = _seg_i(i)
        return (bi, hi, 0)

    acc = pl.pallas_call(
        _dense_body,
        grid=(_CLS_STEPS,),
        in_specs=[
            pl.BlockSpec((_CLS_RB2, 128), lambda i: (i, 0)),
            pl.BlockSpec((128, 8), lambda i: (0, 0)),
            pl.BlockSpec((1, _NCLS, _SEG_BH, _SEG_H), _seg_map),
            pl.BlockSpec((1, _SEG_BH, _SEG_H), _lab_map),
        ],
        out_specs=pl.BlockSpec((1, 2), lambda i: (0, 0),
                               memory_space=pltpu.SMEM),
        out_shape=jax.ShapeDtypeStruct((1, 2), jnp.float32),
    )(clsp, jnp.asarray(_M8, dtype=jnp.bfloat16), Seg,
      seg_label.astype(jnp.int32))
    cls_dense = acc[0, 0]
    seg_sum = acc[0, 1]

    idxt = jnp.transpose(idx_batch[..., 1:].astype(jnp.int32), (2, 0, 1))
    gtt = jnp.transpose(gt_box_batch, (2, 0, 1))
    dft = jnp.transpose(df_box_batch, (2, 0, 1))
    mesh = plsc.VectorSubcoreMesh(core_axis_name="c", subcore_axis_name="s")
    parts = pl.kernel(
        _sc_body,
        mesh=mesh,
        compiler_params=pltpu.CompilerParams(needs_layout_passes=False,
                                             use_tc_tiling_on_sc=True),
        out_type=jax.ShapeDtypeStruct((_B, 16), jnp.float32),
        scratch_types=[
            pltpu.VMEM((_NOBJ,), jnp.int32),
            pltpu.VMEM((_NOBJ,), jnp.int32),
            pltpu.VMEM((_NOBJ,), jnp.int32),
            pltpu.VMEM((_NOBJ,), jnp.int32),
            pltpu.VMEM((4, _NOBJ), jnp.float32),
            pltpu.VMEM((4, _NOBJ), jnp.float32),
            pltpu.VMEM((4, 16, 128), jnp.float32),
            pltpu.VMEM((4, 16, 128), jnp.float32),
            pltpu.VMEM((16,), jnp.float32),
            pltpu.SemaphoreType.DMA,
        ],
    )(clsp, locp, idxt, cls_batch.astype(jnp.int32), gtt, dft)

    cls_corr = jnp.sum(parts[:, 0])
    loc_sum = jnp.sum(parts[:, 1])
    npos = jnp.sum(parts[:, 2])

    cls_loss = (cls_dense + cls_corr) / jnp.float32(_TOTAL)
    loc_loss = loc_sum / jnp.maximum(npos, 1.0)
    seg_loss = seg_sum / jnp.float32(_B * _SEG_H * _SEG_H)
    return cls_loss + loc_loss + seg_loss
